# baseline (speedup 1.0000x reference)
import functools

import jax
import jax.numpy as jnp
from jax import lax
from jax.experimental import pallas as pl
from jax.experimental.pallas import tpu as pltpu
from jax.experimental.pallas import tpu_sc as plsc

N = 10000
D = 128
E = 320000
NC = 2
NS = 16
NW = NC * NS
EPW = E // NW
K = 40
NB = EPW // K
NP = N
RPT = NP // NS
DZ = 25


def _sc_agg_build(with_deg):
  mesh = plsc.VectorSubcoreMesh(core_axis_name="c", subcore_axis_name="s")
  out_type = [jax.ShapeDtypeStruct((NC, NP, D), jnp.float32)]
  scratch = [
      pltpu.VMEM((NB, K), jnp.int32),
      pltpu.VMEM((NB, K), jnp.int32),
      pltpu.VMEM((K, D), jnp.float32),
      pltpu.VMEM((K, D), jnp.float32),
      pltpu.VMEM_SHARED((NP, D), jnp.float32),
      pltpu.SemaphoreType.DMA,
      pltpu.SemaphoreType.DMA,
      pltpu.SemaphoreType.DMA,
      pltpu.SemaphoreType.DMA,
      pltpu.SemaphoreType.DMA,
      pltpu.SemaphoreType.DMA,
  ]
  if with_deg:
    out_type.append(jax.ShapeDtypeStruct((NC, NP, 16), jnp.float32))
    scratch += [
        pltpu.VMEM((K, 16), jnp.float32),
        pltpu.VMEM_SHARED((NP, 16), jnp.float32),
        pltpu.VMEM((DZ, 16), jnp.float32),
    ]

  def body(p_hbm, src_hbm, dst_hbm, *rest):
    if with_deg:
      (acc_o, deg_o, src_v, dst_v, rows_v, rows2_v, acc_sh, sem, sem2,
       ssem, ssem2, osem, osem2, ones_v, deg_sh, dzb_v) = rest
    else:
      (acc_o, src_v, dst_v, rows_v, rows2_v, acc_sh, sem, sem2,
       ssem, ssem2, osem, osem2) = rest
    cid = lax.axis_index("c")
    sid = lax.axis_index("s")
    wid = cid * NS + sid

    def zfill(i, _):
      for c in range(D // 16):
        rows_v[i, pl.ds(c * 16, 16)] = jnp.zeros((16,), jnp.float32)
      return 0
    lax.fori_loop(0, K, zfill, 0)
    for j in range(RPT // K):
      pltpu.sync_copy(rows_v, acc_sh.at[pl.ds(sid * RPT + j * K, K)])
    rem = RPT % K
    if rem:
      pltpu.sync_copy(rows_v.at[pl.ds(0, rem)],
                      acc_sh.at[pl.ds(sid * RPT + (RPT // K) * K, rem)])
    if with_deg:
      def dzfill(i, _):
        dzb_v[i, :] = jnp.zeros((16,), jnp.float32)
        return 0
      lax.fori_loop(0, DZ, dzfill, 0)
      for j in range(RPT // DZ):
        pltpu.sync_copy(dzb_v, deg_sh.at[pl.ds(sid * RPT + j * DZ, DZ)])
      def ofill(i, _):
        ones_v[i, :] = jnp.ones((16,), jnp.float32)
        return 0
      lax.fori_loop(0, K, ofill, 0)
    plsc.subcore_barrier()

    pltpu.sync_copy(src_hbm.at[wid], src_v)
    pltpu.sync_copy(dst_hbm.at[wid], dst_v)

    bufs = ((rows_v, sem, ssem, osem), (rows2_v, sem2, ssem2, osem2))

    def start(i, b):
      pltpu.async_copy(p_hbm.at[src_v.at[i]], bufs[b][0], bufs[b][1])

    def wait_gather(b):
      pltpu.make_async_copy(p_hbm.at[src_v.at[0]], bufs[b][0],
                            bufs[b][1]).wait()

    def issue_scatter(i, b):
      descs = [pltpu.async_copy(bufs[b][0], acc_sh.at[dst_v.at[i]],
                                bufs[b][2], add=True)]
      if with_deg:
        descs.append(pltpu.async_copy(ones_v, deg_sh.at[dst_v.at[i]],
                                      bufs[b][3], add=True))
      return descs

    def finish(i, b):
      wait_gather(b)
      pltpu.sync_copy(bufs[b][0], acc_sh.at[dst_v.at[i]], add=True)
      if with_deg:
        pltpu.sync_copy(ones_v, deg_sh.at[dst_v.at[i]], add=True)

    start(0, 0)
    start(1, 1)

    def step(g, _):
      i0 = 2 * g
      wait_gather(0)
      d0 = issue_scatter(i0, 0)
      wait_gather(1)
      d1 = issue_scatter(i0 + 1, 1)
      for d in d0:
        d.wait()
      start(i0 + 2, 0)
      for d in d1:
        d.wait()
      start(i0 + 3, 1)
      return 0
    lax.fori_loop(0, (NB - 2) // 2, step, 0)
    finish(NB - 2, 0)
    finish(NB - 1, 1)
    plsc.subcore_barrier()

    r0 = sid * RPT
    pltpu.sync_copy(acc_sh.at[pl.ds(r0, RPT)],
                    acc_o.at[cid, pl.ds(r0, RPT)])
    if with_deg:
      pltpu.sync_copy(deg_sh.at[pl.ds(r0, RPT)],
                      deg_o.at[cid, pl.ds(r0, RPT)])

  return pl.kernel(
      body, out_type=out_type, mesh=mesh, scratch_types=scratch,
      compiler_params=pltpu.CompilerParams(use_tc_tiling_on_sc=False))


_sc_agg_deg = _sc_agg_build(True)
_sc_agg = _sc_agg_build(False)


def _dot(a, b):
  return jnp.dot(a, b, preferred_element_type=jnp.float32,
                 precision=lax.Precision.HIGHEST)


def _layer1_body(x_ref, a0_ref, a1_ref, deg_ref, ws_ref, wn_ref,
                 b_ref, o_ref, inv_ref):
  deg = jnp.sum(deg_ref[0] + deg_ref[1], axis=1, keepdims=True) * (1.0 / 16.0)
  inv = 1.0 / jnp.maximum(deg, 1.0)
  agg = (a0_ref[0] + a1_ref[0]) * inv
  y = _dot(x_ref[...], ws_ref[...]) + _dot(agg, wn_ref[...]) + b_ref[...]
  o_ref[...] = jnp.maximum(y, 0.0)
  inv_ref[...] = inv


def _layer2_body(h_ref, a0_ref, a1_ref, inv_ref, ws_ref, wn_ref, b_ref,
                 o_ref):
  agg = (a0_ref[0] + a1_ref[0]) * inv_ref[...]
  y = _dot(h_ref[...], ws_ref[...]) + _dot(agg, wn_ref[...]) + b_ref[...]
  y = jnp.maximum(y, 0.0)
  nrm = jnp.sqrt(jnp.sum(y * y, axis=1, keepdims=True))
  o_ref[...] = y / jnp.maximum(nrm, 1e-12)


_BR = 1000
_row = pl.BlockSpec((_BR, D), lambda i: (i, 0))
_col1 = pl.BlockSpec((_BR, 1), lambda i: (i, 0))
_acc0 = pl.BlockSpec((1, _BR, D), lambda i: (0, i, 0))
_acc1 = pl.BlockSpec((1, _BR, D), lambda i: (1, i, 0))
_degs = pl.BlockSpec((2, _BR, 16), lambda i: (0, i, 0))
_wspec = pl.BlockSpec((D, D), lambda i: (0, 0))
_bspec = pl.BlockSpec((1, D), lambda i: (0, 0))

_tc_layer1 = pl.pallas_call(
    _layer1_body,
    grid=(N // _BR,),
    in_specs=[_row, _acc0, _acc1, _degs, _wspec, _wspec, _bspec],
    out_specs=[_row, _col1],
    out_shape=[jax.ShapeDtypeStruct((N, D), jnp.float32),
               jax.ShapeDtypeStruct((N, 1), jnp.float32)],
)

_tc_layer2 = pl.pallas_call(
    _layer2_body,
    grid=(N // _BR,),
    in_specs=[_row, _acc0, _acc1, _col1, _wspec, _wspec, _bspec],
    out_specs=_row,
    out_shape=jax.ShapeDtypeStruct((N, D), jnp.float32),
)


@jax.jit
def kernel(x, edge_index, W_self1, W_neigh1, b1, W_self2, W_neigh2, b2):
  src = edge_index[0].astype(jnp.int32).reshape(NW, NB, K)
  dst = edge_index[1].astype(jnp.int32).reshape(NW, NB, K)
  acc1, degp = _sc_agg_deg(x, src, dst)
  h1, inv = _tc_layer1(x, acc1, acc1, degp,
                       W_self1, W_neigh1, b1.reshape(1, D))
  (acc2,) = _sc_agg(h1, src, dst)
  return _tc_layer2(h1, acc2, acc2, inv,
                    W_self2, W_neigh2, b2.reshape(1, D))


# speedup vs baseline: 1.2093x; 1.2093x over previous
import functools

import jax
import jax.numpy as jnp
from jax import lax
from jax.experimental import pallas as pl
from jax.experimental.pallas import tpu as pltpu
from jax.experimental.pallas import tpu_sc as plsc

N = 10000
D = 128
E = 320000
NC = 2
NS = 16
NW = NC * NS
EPW = E // NW
K = 40
NB = EPW // K
NP = N
RPT = NP // NS
DZ = 25


def _sc_agg_build(with_deg, k):
  nb = EPW // k
  assert nb * k == EPW and k % 8 == 0 and k <= 128
  mesh = plsc.VectorSubcoreMesh(core_axis_name="c", subcore_axis_name="s")
  out_type = [jax.ShapeDtypeStruct((NC, NP, D), jnp.float32)]
  scratch = [
      pltpu.VMEM((nb, k), jnp.int32),
      pltpu.VMEM((nb, k), jnp.int32),
      pltpu.VMEM((k, D), jnp.float32),
      pltpu.VMEM((k, D), jnp.float32),
      pltpu.VMEM_SHARED((NP, D), jnp.float32),
      pltpu.SemaphoreType.DMA,
      pltpu.SemaphoreType.DMA,
      pltpu.SemaphoreType.DMA,
  ]
  if with_deg:
    out_type.append(jax.ShapeDtypeStruct((NC, NP, 16), jnp.float32))
    scratch += [
        pltpu.VMEM((k, 16), jnp.float32),
        pltpu.VMEM_SHARED((NP, 16), jnp.float32),
        pltpu.VMEM((DZ, 16), jnp.float32),
    ]

  def body(p_hbm, src_hbm, dst_hbm, *rest):
    if with_deg:
      (acc_o, deg_o, src_v, dst_v, rows_v, rows2_v, acc_sh, sem, sem2,
       osem, ones_v, deg_sh, dzb_v) = rest
    else:
      (acc_o, src_v, dst_v, rows_v, rows2_v, acc_sh, sem, sem2, osem) = rest
    cid = lax.axis_index("c")
    sid = lax.axis_index("s")
    wid = cid * NS + sid

    def zfill(i, _):
      for c in range(D // 16):
        rows_v[i, pl.ds(c * 16, 16)] = jnp.zeros((16,), jnp.float32)
      return 0
    lax.fori_loop(0, k, zfill, 0)
    for j in range(RPT // k):
      pltpu.sync_copy(rows_v, acc_sh.at[pl.ds(sid * RPT + j * k, k)])
    rem = RPT % k
    if rem:
      pltpu.sync_copy(rows_v.at[pl.ds(0, rem)],
                      acc_sh.at[pl.ds(sid * RPT + (RPT // k) * k, rem)])
    if with_deg:
      def dzfill(i, _):
        dzb_v[i, :] = jnp.zeros((16,), jnp.float32)
        return 0
      lax.fori_loop(0, DZ, dzfill, 0)
      for j in range(RPT // DZ):
        pltpu.sync_copy(dzb_v, deg_sh.at[pl.ds(sid * RPT + j * DZ, DZ)])
      def ofill(i, _):
        ones_v[i, :] = jnp.ones((16,), jnp.float32)
        return 0
      lax.fori_loop(0, k, ofill, 0)
    plsc.subcore_barrier()

    pltpu.sync_copy(src_hbm.at[wid], src_v)
    pltpu.sync_copy(dst_hbm.at[wid], dst_v)

    bufs = ((rows_v, sem), (rows2_v, sem2))

    def start(i, b):
      pltpu.async_copy(p_hbm.at[src_v.at[i]], bufs[b][0], bufs[b][1])

    def finish(i, b):
      pltpu.make_async_copy(p_hbm.at[src_v.at[0]], bufs[b][0],
                            bufs[b][1]).wait()
      pltpu.sync_copy(bufs[b][0], acc_sh.at[dst_v.at[i]], add=True)
      if with_deg:
        pltpu.async_copy(ones_v, deg_sh.at[dst_v.at[i]], osem, add=True)

    if nb % 2 == 0:
      start(0, 0)
      start(1, 1)

      def step(g, _):
        i0 = 2 * g
        finish(i0, 0)
        start(i0 + 2, 0)
        finish(i0 + 1, 1)
        start(i0 + 3, 1)
        return 0
      lax.fori_loop(0, (nb - 2) // 2, step, 0)
      finish(nb - 2, 0)
      finish(nb - 1, 1)
    else:
      start(0, 0)

      def step(g, _):
        i0 = 2 * g
        start(i0 + 1, 1)
        finish(i0, 0)
        start(i0 + 2, 0)
        finish(i0 + 1, 1)
        return 0
      lax.fori_loop(0, (nb - 1) // 2, step, 0)
      finish(nb - 1, 0)
    if with_deg:
      def odrain(i, _):
        pltpu.make_async_copy(ones_v, deg_sh.at[dst_v.at[0]], osem).wait()
        return 0
      lax.fori_loop(0, nb, odrain, 0)
    plsc.subcore_barrier()

    r0 = sid * RPT
    pltpu.sync_copy(acc_sh.at[pl.ds(r0, RPT)],
                    acc_o.at[cid, pl.ds(r0, RPT)])
    if with_deg:
      pltpu.sync_copy(deg_sh.at[pl.ds(r0, RPT)],
                      deg_o.at[cid, pl.ds(r0, RPT)])

  return pl.kernel(
      body, out_type=out_type, mesh=mesh, scratch_types=scratch,
      compiler_params=pltpu.CompilerParams(use_tc_tiling_on_sc=False))


K1 = 40
K2 = 80
_sc_agg_deg = _sc_agg_build(True, K1)
_sc_agg = _sc_agg_build(False, K2)


def _dot(a, b):
  return jnp.dot(a, b, preferred_element_type=jnp.float32,
                 precision=lax.Precision.HIGHEST)


def _layer1_body(x_ref, a0_ref, a1_ref, deg_ref, ws_ref, wn_ref,
                 b_ref, o_ref, inv_ref):
  deg = jnp.sum(deg_ref[0] + deg_ref[1], axis=1, keepdims=True) * (1.0 / 16.0)
  inv = 1.0 / jnp.maximum(deg, 1.0)
  agg = (a0_ref[0] + a1_ref[0]) * inv
  y = _dot(x_ref[...], ws_ref[...]) + _dot(agg, wn_ref[...]) + b_ref[...]
  o_ref[...] = jnp.maximum(y, 0.0)
  inv_ref[...] = inv


def _layer2_body(h_ref, a0_ref, a1_ref, inv_ref, ws_ref, wn_ref, b_ref,
                 o_ref):
  agg = (a0_ref[0] + a1_ref[0]) * inv_ref[...]
  y = _dot(h_ref[...], ws_ref[...]) + _dot(agg, wn_ref[...]) + b_ref[...]
  y = jnp.maximum(y, 0.0)
  nrm = jnp.sqrt(jnp.sum(y * y, axis=1, keepdims=True))
  o_ref[...] = y / jnp.maximum(nrm, 1e-12)


_BR = 1000
_row = pl.BlockSpec((_BR, D), lambda i: (i, 0))
_col1 = pl.BlockSpec((_BR, 1), lambda i: (i, 0))
_acc0 = pl.BlockSpec((1, _BR, D), lambda i: (0, i, 0))
_acc1 = pl.BlockSpec((1, _BR, D), lambda i: (1, i, 0))
_degs = pl.BlockSpec((2, _BR, 16), lambda i: (0, i, 0))
_wspec = pl.BlockSpec((D, D), lambda i: (0, 0))
_bspec = pl.BlockSpec((1, D), lambda i: (0, 0))

_tc_layer1 = pl.pallas_call(
    _layer1_body,
    grid=(N // _BR,),
    in_specs=[_row, _acc0, _acc1, _degs, _wspec, _wspec, _bspec],
    out_specs=[_row, _col1],
    out_shape=[jax.ShapeDtypeStruct((N, D), jnp.float32),
               jax.ShapeDtypeStruct((N, 1), jnp.float32)],
)

_tc_layer2 = pl.pallas_call(
    _layer2_body,
    grid=(N // _BR,),
    in_specs=[_row, _acc0, _acc1, _col1, _wspec, _wspec, _bspec],
    out_specs=_row,
    out_shape=jax.ShapeDtypeStruct((N, D), jnp.float32),
)


@jax.jit
def kernel(x, edge_index, W_self1, W_neigh1, b1, W_self2, W_neigh2, b2):
  src = edge_index[0].astype(jnp.int32)
  dst = edge_index[1].astype(jnp.int32)
  acc1, degp = _sc_agg_deg(x, src.reshape(NW, EPW // K1, K1),
                           dst.reshape(NW, EPW // K1, K1))
  h1, inv = _tc_layer1(x, acc1, acc1, degp,
                       W_self1, W_neigh1, b1.reshape(1, D))
  (acc2,) = _sc_agg(h1, src.reshape(NW, EPW // K2, K2),
                    dst.reshape(NW, EPW // K2, K2))
  return _tc_layer2(h1, acc2, acc2, inv,
                    W_self2, W_neigh2, b2.reshape(1, D))


# speedup vs baseline: 1.3594x; 1.1241x over previous
import functools

import jax
import jax.numpy as jnp
from jax import lax
from jax.experimental import pallas as pl
from jax.experimental.pallas import tpu as pltpu
from jax.experimental.pallas import tpu_sc as plsc

N = 10000
D = 128
E = 320000
NC = 2
NS = 16
NW = NC * NS
EPW = E // NW
K = 40
NB = EPW // K
NP = N
RPT = NP // NS
DZ = 25


def _sc_agg_build(with_deg, k):
  nb = EPW // k
  assert nb * k == EPW and k % 16 == 0 and k <= 128 and nb % 2 == 1
  mesh = plsc.VectorSubcoreMesh(core_axis_name="c", subcore_axis_name="s")
  out_type = [jax.ShapeDtypeStruct((NC, NP, D), jnp.float32)]
  scratch = [
      pltpu.VMEM((nb, k), jnp.int32),
      pltpu.VMEM((4, k), jnp.int32),
      pltpu.VMEM((4, k), jnp.int32),
      pltpu.VMEM((k, D), jnp.float32),
      pltpu.VMEM((k, D), jnp.float32),
      pltpu.VMEM_SHARED((NP, D), jnp.float32),
      pltpu.SemaphoreType.DMA,
      pltpu.SemaphoreType.DMA,
      pltpu.SemaphoreType.DMA,
  ]
  if with_deg:
    out_type.append(jax.ShapeDtypeStruct((NC, NP, 16), jnp.float32))
    scratch += [
        pltpu.VMEM((k, 16), jnp.float32),
        pltpu.VMEM_SHARED((NP, 16), jnp.float32),
        pltpu.VMEM((DZ, 16), jnp.float32),
    ]

  def body(p_hbm, pidx_hbm, *rest):
    if with_deg:
      (acc_o, deg_o, pidx_v, sidx_v, didx_v, rows_v, rows2_v, acc_sh,
       sem, sem2, osem, ones_v, deg_sh, dzb_v) = rest
    else:
      (acc_o, pidx_v, sidx_v, didx_v, rows_v, rows2_v, acc_sh,
       sem, sem2, osem) = rest
    cid = lax.axis_index("c")
    sid = lax.axis_index("s")
    wid = cid * NS + sid

    def zfill(i, _):
      for c in range(D // 16):
        rows_v[i, pl.ds(c * 16, 16)] = jnp.zeros((16,), jnp.float32)
      return 0
    lax.fori_loop(0, k, zfill, 0)
    for j in range(RPT // k):
      pltpu.sync_copy(rows_v, acc_sh.at[pl.ds(sid * RPT + j * k, k)])
    rem = RPT % k
    if rem:
      pltpu.sync_copy(rows_v.at[pl.ds(0, rem)],
                      acc_sh.at[pl.ds(sid * RPT + (RPT // k) * k, rem)])
    if with_deg:
      def dzfill(i, _):
        dzb_v[i, :] = jnp.zeros((16,), jnp.float32)
        return 0
      lax.fori_loop(0, DZ, dzfill, 0)
      for j in range(RPT // DZ):
        pltpu.sync_copy(dzb_v, deg_sh.at[pl.ds(sid * RPT + j * DZ, DZ)])
      def ofill(i, _):
        ones_v[i, :] = jnp.ones((16,), jnp.float32)
        return 0
      lax.fori_loop(0, k, ofill, 0)
    plsc.subcore_barrier()

    pltpu.sync_copy(pidx_hbm.at[wid], pidx_v)

    bufs = ((rows_v, sem), (rows2_v, sem2))

    def unpack(i, s):
      for c in range(k // 16):
        v = pidx_v[i, pl.ds(c * 16, 16)]
        sidx_v[s, pl.ds(c * 16, 16)] = v & 0xFFFF
        didx_v[s, pl.ds(c * 16, 16)] = lax.shift_right_logical(v, 16)

    def start(b, s):
      pltpu.async_copy(p_hbm.at[sidx_v.at[s]], bufs[b][0], bufs[b][1])

    def finish(b, s, nxt=None, nxt_s=None):
      pltpu.make_async_copy(p_hbm.at[sidx_v.at[s]], bufs[b][0],
                            bufs[b][1]).wait()
      pltpu.sync_copy(bufs[b][0], acc_sh.at[didx_v.at[s]], add=True)
      if with_deg:
        pltpu.async_copy(ones_v, deg_sh.at[didx_v.at[s]], osem, add=True)
      if nxt is not None:
        unpack(nxt, nxt_s)
        start(b, nxt_s)

    assert nb % 4 == 1
    unpack(0, 0)
    start(0, 0)
    unpack(1, 1)
    start(1, 1)

    def step(g, _):
      i0 = 4 * g
      finish(0, 0, i0 + 2, 2)
      finish(1, 1, i0 + 3, 3)
      finish(0, 2, i0 + 4, 0)
      finish(1, 3, i0 + 5, 1)
      return 0
    lax.fori_loop(0, (nb - 5) // 4, step, 0)
    finish(0, 0, nb - 3, 2)
    finish(1, 1, nb - 2, 3)
    finish(0, 2, nb - 1, 0)
    finish(1, 3)
    finish(0, 0)
    if with_deg:
      def odrain(i, _):
        pltpu.make_async_copy(ones_v, deg_sh.at[didx_v.at[0]], osem).wait()
        return 0
      lax.fori_loop(0, nb, odrain, 0)
    plsc.subcore_barrier()

    r0 = sid * RPT
    pltpu.sync_copy(acc_sh.at[pl.ds(r0, RPT)],
                    acc_o.at[cid, pl.ds(r0, RPT)])
    if with_deg:
      pltpu.sync_copy(deg_sh.at[pl.ds(r0, RPT)],
                      deg_o.at[cid, pl.ds(r0, RPT)])

  return pl.kernel(
      body, out_type=out_type, mesh=mesh, scratch_types=scratch,
      compiler_params=pltpu.CompilerParams(use_tc_tiling_on_sc=False))


KB = 80
_sc_agg_deg = _sc_agg_build(True, KB)
_sc_agg = _sc_agg_build(False, KB)


def _dot(a, b):
  return jnp.dot(a, b, preferred_element_type=jnp.float32,
                 precision=lax.Precision.HIGHEST)


def _layer1_body(x_ref, a0_ref, a1_ref, deg_ref, ws_ref, wn_ref,
                 b_ref, o_ref, inv_ref):
  deg = jnp.sum(deg_ref[0] + deg_ref[1], axis=1, keepdims=True) * (1.0 / 16.0)
  inv = 1.0 / jnp.maximum(deg, 1.0)
  agg = (a0_ref[0] + a1_ref[0]) * inv
  y = _dot(x_ref[...], ws_ref[...]) + _dot(agg, wn_ref[...]) + b_ref[...]
  o_ref[...] = jnp.maximum(y, 0.0)
  inv_ref[...] = inv


def _layer2_body(h_ref, a0_ref, a1_ref, inv_ref, ws_ref, wn_ref, b_ref,
                 o_ref):
  agg = (a0_ref[0] + a1_ref[0]) * inv_ref[...]
  y = _dot(h_ref[...], ws_ref[...]) + _dot(agg, wn_ref[...]) + b_ref[...]
  y = jnp.maximum(y, 0.0)
  nrm = jnp.sqrt(jnp.sum(y * y, axis=1, keepdims=True))
  o_ref[...] = y / jnp.maximum(nrm, 1e-12)


_BR = 1000
_row = pl.BlockSpec((_BR, D), lambda i: (i, 0))
_col1 = pl.BlockSpec((_BR, 1), lambda i: (i, 0))
_acc0 = pl.BlockSpec((1, _BR, D), lambda i: (0, i, 0))
_acc1 = pl.BlockSpec((1, _BR, D), lambda i: (1, i, 0))
_degs = pl.BlockSpec((2, _BR, 16), lambda i: (0, i, 0))
_wspec = pl.BlockSpec((D, D), lambda i: (0, 0))
_bspec = pl.BlockSpec((1, D), lambda i: (0, 0))

_tc_layer1 = pl.pallas_call(
    _layer1_body,
    grid=(N // _BR,),
    in_specs=[_row, _acc0, _acc1, _degs, _wspec, _wspec, _bspec],
    out_specs=[_row, _col1],
    out_shape=[jax.ShapeDtypeStruct((N, D), jnp.float32),
               jax.ShapeDtypeStruct((N, 1), jnp.float32)],
)

_tc_layer2 = pl.pallas_call(
    _layer2_body,
    grid=(N // _BR,),
    in_specs=[_row, _acc0, _acc1, _col1, _wspec, _wspec, _bspec],
    out_specs=_row,
    out_shape=jax.ShapeDtypeStruct((N, D), jnp.float32),
)


@jax.jit
def kernel(x, edge_index, W_self1, W_neigh1, b1, W_self2, W_neigh2, b2):
  src = edge_index[0].astype(jnp.int32)
  dst = edge_index[1].astype(jnp.int32)
  pidx = (src | (dst << 16)).reshape(NW, EPW // KB, KB)
  acc1, degp = _sc_agg_deg(x, pidx)
  h1, inv = _tc_layer1(x, acc1, acc1, degp,
                       W_self1, W_neigh1, b1.reshape(1, D))
  (acc2,) = _sc_agg(h1, pidx)
  return _tc_layer2(h1, acc2, acc2, inv,
                    W_self2, W_neigh2, b2.reshape(1, D))


# speedup vs baseline: 1.6669x; 1.2262x over previous
import functools

import jax
import jax.numpy as jnp
from jax import lax
from jax.experimental import pallas as pl
from jax.experimental.pallas import tpu as pltpu
from jax.experimental.pallas import tpu_sc as plsc

N = 10000
D = 128
E = 320000
NC = 2
NS = 16
NW = NC * NS
EPW = E // NW
K = 40
NB = EPW // K
NP = N
RPT = NP // NS
DZ = 25


def _sc_agg_build(with_deg, k, dtype):
  nb = EPW // k
  assert nb * k == EPW and k % 16 == 0 and k <= 128 and D % 32 == 0
  RB = 4 if dtype == jnp.bfloat16 else 2
  IS = 8
  lanes = 16 if dtype == jnp.float32 else 32
  mesh = plsc.VectorSubcoreMesh(core_axis_name="c", subcore_axis_name="s")
  out_type = [jax.ShapeDtypeStruct((NC, NP, D), dtype)]
  scratch = [
      pltpu.VMEM((nb, k), jnp.int32),
      pltpu.VMEM((IS, k), jnp.int32),
      pltpu.VMEM((IS, k), jnp.int32),
  ] + [pltpu.VMEM((k, D), dtype) for _ in range(RB)] + [
      pltpu.VMEM_SHARED((NP, D), dtype),
  ] + [pltpu.SemaphoreType.DMA for _ in range(RB)] + [
      pltpu.SemaphoreType.DMA,
  ]
  if with_deg:
    out_type.append(jax.ShapeDtypeStruct((NC, NP, 16), jnp.float32))
    scratch += [
        pltpu.VMEM((k, 16), jnp.float32),
        pltpu.VMEM_SHARED((NP, 16), jnp.float32),
        pltpu.VMEM((DZ, 16), jnp.float32),
    ]

  def body(p_hbm, pidx_hbm, *rest):
    rest = list(rest)
    acc_o = rest.pop(0)
    if with_deg:
      deg_o = rest.pop(0)
    pidx_v, sidx_v, didx_v = rest[0], rest[1], rest[2]
    rows = rest[3:3 + RB]
    acc_sh = rest[3 + RB]
    sems = rest[4 + RB:4 + 2 * RB]
    osem = rest[4 + 2 * RB]
    if with_deg:
      ones_v, deg_sh, dzb_v = rest[5 + 2 * RB:8 + 2 * RB]
    cid = lax.axis_index("c")
    sid = lax.axis_index("s")
    wid = cid * NS + sid

    rows_v = rows[0]
    def zfill(i, _):
      for c in range(D // lanes):
        rows_v[i, pl.ds(c * lanes, lanes)] = jnp.zeros((lanes,), dtype)
      return 0
    lax.fori_loop(0, k, zfill, 0)
    for j in range(RPT // k):
      pltpu.sync_copy(rows_v, acc_sh.at[pl.ds(sid * RPT + j * k, k)])
    rem = RPT % k
    if rem:
      pltpu.sync_copy(rows_v.at[pl.ds(0, rem)],
                      acc_sh.at[pl.ds(sid * RPT + (RPT // k) * k, rem)])
    if with_deg:
      def dzfill(i, _):
        dzb_v[i, :] = jnp.zeros((16,), jnp.float32)
        return 0
      lax.fori_loop(0, DZ, dzfill, 0)
      for j in range(RPT // DZ):
        pltpu.sync_copy(dzb_v, deg_sh.at[pl.ds(sid * RPT + j * DZ, DZ)])
      def ofill(i, _):
        ones_v[i, :] = jnp.ones((16,), jnp.float32)
        return 0
      lax.fori_loop(0, k, ofill, 0)
    plsc.subcore_barrier()

    pltpu.sync_copy(pidx_hbm.at[wid], pidx_v)


    def unpack(i, s):
      for c in range(k // 16):
        v = pidx_v[i, pl.ds(c * 16, 16)]
        sidx_v[s, pl.ds(c * 16, 16)] = v & 0xFFFF
        didx_v[s, pl.ds(c * 16, 16)] = lax.shift_right_logical(v, 16)

    def start(b, s):
      pltpu.async_copy(p_hbm.at[sidx_v.at[s]], rows[b], sems[b])

    def finish(b, s, nxt=None, nxt_s=None):
      pltpu.make_async_copy(p_hbm.at[sidx_v.at[s]], rows[b], sems[b]).wait()
      pltpu.sync_copy(rows[b], acc_sh.at[didx_v.at[s]], add=True)
      if with_deg:
        pltpu.async_copy(ones_v, deg_sh.at[didx_v.at[s]], osem, add=True)
      if nxt is not None:
        unpack(nxt, nxt_s)
        start(b, nxt_s)

    assert IS % RB == 0 and nb > IS
    G = (nb - RB) // IS
    assert nb - IS * G <= IS
    for i in range(RB):
      unpack(i, i)
      start(i, i)

    def step(g, _):
      i0 = IS * g
      for j in range(IS):
        finish(j % RB, j, i0 + j + RB, (j + RB) % IS)
      return 0
    lax.fori_loop(0, G, step, 0)
    for i in range(IS * G, nb):
      j = i - IS * G
      if i + RB < nb:
        finish(j % RB, j % IS, i + RB, (j + RB) % IS)
      else:
        finish(j % RB, j % IS)
    if with_deg:
      def odrain(i, _):
        pltpu.make_async_copy(ones_v, deg_sh.at[didx_v.at[0]], osem).wait()
        return 0
      lax.fori_loop(0, nb, odrain, 0)
    plsc.subcore_barrier()

    r0 = sid * RPT
    pltpu.sync_copy(acc_sh.at[pl.ds(r0, RPT)],
                    acc_o.at[cid, pl.ds(r0, RPT)])
    if with_deg:
      pltpu.sync_copy(deg_sh.at[pl.ds(r0, RPT)],
                      deg_o.at[cid, pl.ds(r0, RPT)])

  return pl.kernel(
      body, out_type=out_type, mesh=mesh, scratch_types=scratch,
      compiler_params=pltpu.CompilerParams(use_tc_tiling_on_sc=False))


KB = 80
_sc_agg_deg = _sc_agg_build(True, KB, jnp.bfloat16)
_sc_agg = _sc_agg_build(False, KB, jnp.bfloat16)


def _dot(a, b):
  return jnp.dot(a, b, preferred_element_type=jnp.float32,
                 precision=lax.Precision.HIGHEST)


def _layer1_body(x_ref, a0_ref, a1_ref, deg_ref, ws_ref, wn_ref,
                 b_ref, o_ref, ob_ref, inv_ref):
  deg = jnp.sum(deg_ref[0] + deg_ref[1], axis=1, keepdims=True) * (1.0 / 16.0)
  inv = 1.0 / jnp.maximum(deg, 1.0)
  agg = (a0_ref[0].astype(jnp.float32) + a1_ref[0].astype(jnp.float32)) * inv
  y = _dot(x_ref[...], ws_ref[...]) + _dot(agg, wn_ref[...]) + b_ref[...]
  h = jnp.maximum(y, 0.0)
  o_ref[...] = h
  ob_ref[...] = h.astype(jnp.bfloat16)
  inv_ref[...] = inv


def _layer2_body(h_ref, a0_ref, a1_ref, inv_ref, ws_ref, wn_ref, b_ref,
                 o_ref):
  agg = (a0_ref[0].astype(jnp.float32)
         + a1_ref[0].astype(jnp.float32)) * inv_ref[...]
  y = _dot(h_ref[...], ws_ref[...]) + _dot(agg, wn_ref[...]) + b_ref[...]
  y = jnp.maximum(y, 0.0)
  nrm = jnp.sqrt(jnp.sum(y * y, axis=1, keepdims=True))
  o_ref[...] = y / jnp.maximum(nrm, 1e-12)


_BR = 1000
_row = pl.BlockSpec((_BR, D), lambda i: (i, 0))
_col1 = pl.BlockSpec((_BR, 1), lambda i: (i, 0))
_acc0 = pl.BlockSpec((1, _BR, D), lambda i: (0, i, 0))
_acc1 = pl.BlockSpec((1, _BR, D), lambda i: (1, i, 0))
_degs = pl.BlockSpec((2, _BR, 16), lambda i: (0, i, 0))
_wspec = pl.BlockSpec((D, D), lambda i: (0, 0))
_bspec = pl.BlockSpec((1, D), lambda i: (0, 0))

_tc_layer1 = pl.pallas_call(
    _layer1_body,
    grid=(N // _BR,),
    in_specs=[_row, _acc0, _acc1, _degs, _wspec, _wspec, _bspec],
    out_specs=[_row, _row, _col1],
    out_shape=[jax.ShapeDtypeStruct((N, D), jnp.float32),
               jax.ShapeDtypeStruct((N, D), jnp.bfloat16),
               jax.ShapeDtypeStruct((N, 1), jnp.float32)],
)

_tc_layer2 = pl.pallas_call(
    _layer2_body,
    grid=(N // _BR,),
    in_specs=[_row, _acc0, _acc1, _col1, _wspec, _wspec, _bspec],
    out_specs=_row,
    out_shape=jax.ShapeDtypeStruct((N, D), jnp.float32),
)


@jax.jit
def kernel(x, edge_index, W_self1, W_neigh1, b1, W_self2, W_neigh2, b2):
  src = edge_index[0].astype(jnp.int32)
  dst = edge_index[1].astype(jnp.int32)
  pidx = (src | (dst << 16)).reshape(NW, EPW // KB, KB)
  acc1, degp = _sc_agg_deg(x.astype(jnp.bfloat16), pidx)
  h1, h1b, inv = _tc_layer1(x, acc1, acc1, degp,
                            W_self1, W_neigh1, b1.reshape(1, D))
  (acc2,) = _sc_agg(h1b, pidx)
  return _tc_layer2(h1, acc2, acc2, inv,
                    W_self2, W_neigh2, b2.reshape(1, D))


# speedup vs baseline: 1.7260x; 1.0355x over previous
import functools

import jax
import jax.numpy as jnp
from jax import lax
from jax.experimental import pallas as pl
from jax.experimental.pallas import tpu as pltpu
from jax.experimental.pallas import tpu_sc as plsc

N = 10000
D = 128
E = 320000
NC = 2
NS = 16
NW = NC * NS
EPW = E // NW
K = 40
NB = EPW // K
NP = N
RPT = NP // NS
DZ = 25


def _sc_agg_build(with_deg, k, dtype):
  nb = EPW // k
  assert nb * k == EPW and k % 16 == 0 and k <= 128 and D % 32 == 0
  RB = 4 if dtype == jnp.bfloat16 else 2
  IS = 8
  lanes = 16 if dtype == jnp.float32 else 32
  mesh = plsc.VectorSubcoreMesh(core_axis_name="c", subcore_axis_name="s")
  out_type = [jax.ShapeDtypeStruct((NC, NP, D), dtype)]
  scratch = [
      pltpu.VMEM((nb, k), jnp.int32),
      pltpu.VMEM((IS, k), jnp.int32),
      pltpu.VMEM((IS, k), jnp.int32),
  ] + [pltpu.VMEM((k, D), dtype) for _ in range(RB)] + [
      pltpu.VMEM_SHARED((NP, D), dtype),
  ] + [pltpu.SemaphoreType.DMA for _ in range(RB)] + [
      pltpu.SemaphoreType.DMA,
  ]
  if with_deg:
    out_type.append(jax.ShapeDtypeStruct((NC, NP, 16), jnp.float32))
    scratch += [
        pltpu.VMEM((k, 16), jnp.float32),
        pltpu.VMEM_SHARED((NP, 16), jnp.float32),
        pltpu.VMEM((DZ, 16), jnp.float32),
    ]

  def body(p_hbm, pidx_hbm, *rest):
    rest = list(rest)
    acc_o = rest.pop(0)
    if with_deg:
      deg_o = rest.pop(0)
    pidx_v, sidx_v, didx_v = rest[0], rest[1], rest[2]
    rows = rest[3:3 + RB]
    acc_sh = rest[3 + RB]
    sems = rest[4 + RB:4 + 2 * RB]
    osem = rest[4 + 2 * RB]
    if with_deg:
      ones_v, deg_sh, dzb_v = rest[5 + 2 * RB:8 + 2 * RB]
    cid = lax.axis_index("c")
    sid = lax.axis_index("s")
    wid = cid * NS + sid

    rows_v = rows[0]
    def zfill(i, _):
      for c in range(D // lanes):
        rows_v[i, pl.ds(c * lanes, lanes)] = jnp.zeros((lanes,), dtype)
      return 0
    lax.fori_loop(0, k, zfill, 0)
    for j in range(RPT // k):
      pltpu.sync_copy(rows_v, acc_sh.at[pl.ds(sid * RPT + j * k, k)])
    rem = RPT % k
    if rem:
      pltpu.sync_copy(rows_v.at[pl.ds(0, rem)],
                      acc_sh.at[pl.ds(sid * RPT + (RPT // k) * k, rem)])
    if with_deg:
      def dzfill(i, _):
        dzb_v[i, :] = jnp.zeros((16,), jnp.float32)
        return 0
      lax.fori_loop(0, DZ, dzfill, 0)
      for j in range(RPT // DZ):
        pltpu.sync_copy(dzb_v, deg_sh.at[pl.ds(sid * RPT + j * DZ, DZ)])
      def ofill(i, _):
        ones_v[i, :] = jnp.ones((16,), jnp.float32)
        return 0
      lax.fori_loop(0, k, ofill, 0)
    plsc.subcore_barrier()

    pltpu.sync_copy(pidx_hbm.at[wid], pidx_v)


    def unpack(i, s):
      for c in range(k // 16):
        v = pidx_v[i, pl.ds(c * 16, 16)]
        sidx_v[s, pl.ds(c * 16, 16)] = v & 0xFFFF
        didx_v[s, pl.ds(c * 16, 16)] = lax.shift_right_logical(v, 16)

    def start(b, s):
      pltpu.async_copy(p_hbm.at[sidx_v.at[s]], rows[b], sems[b])

    def finish(b, s, nxt=None, nxt_s=None):
      pltpu.make_async_copy(p_hbm.at[sidx_v.at[s]], rows[b], sems[b]).wait()
      pltpu.sync_copy(rows[b], acc_sh.at[didx_v.at[s]], add=True)
      if with_deg:
        pltpu.async_copy(ones_v, deg_sh.at[didx_v.at[s]], osem, add=True)
      if nxt is not None:
        unpack(nxt, nxt_s)
        start(b, nxt_s)

    assert IS % RB == 0 and nb > IS
    G = (nb - RB) // IS
    assert nb - IS * G <= IS
    for i in range(RB):
      unpack(i, i)
      start(i, i)

    def step(g, _):
      i0 = IS * g
      for j in range(IS):
        finish(j % RB, j, i0 + j + RB, (j + RB) % IS)
      return 0
    lax.fori_loop(0, G, step, 0)
    for i in range(IS * G, nb):
      j = i - IS * G
      if i + RB < nb:
        finish(j % RB, j % IS, i + RB, (j + RB) % IS)
      else:
        finish(j % RB, j % IS)
    if with_deg:
      def odrain(i, _):
        pltpu.make_async_copy(ones_v, deg_sh.at[didx_v.at[0]], osem).wait()
        return 0
      lax.fori_loop(0, nb, odrain, 0)
    plsc.subcore_barrier()

    r0 = sid * RPT
    pltpu.sync_copy(acc_sh.at[pl.ds(r0, RPT)],
                    acc_o.at[cid, pl.ds(r0, RPT)])
    if with_deg:
      pltpu.sync_copy(deg_sh.at[pl.ds(r0, RPT)],
                      deg_o.at[cid, pl.ds(r0, RPT)])

  return pl.kernel(
      body, out_type=out_type, mesh=mesh, scratch_types=scratch,
      compiler_params=pltpu.CompilerParams(use_tc_tiling_on_sc=False))


KB = 80
_sc_agg_deg = _sc_agg_build(True, KB, jnp.bfloat16)
_sc_agg = _sc_agg_build(False, KB, jnp.bfloat16)


def _dot(a, b):
  return jnp.dot(a, b, preferred_element_type=jnp.float32,
                 precision=lax.Precision.HIGHEST)


def _self_body(x_ref, w_ref, b_ref, o_ref):
  o_ref[...] = _dot(x_ref[...], w_ref[...]) + b_ref[...]


def _layer1_body(s_ref, a0_ref, a1_ref, deg_ref, wn_ref,
                 o_ref, ob_ref, inv_ref):
  deg = jnp.sum(deg_ref[0] + deg_ref[1], axis=1, keepdims=True) * (1.0 / 16.0)
  inv = 1.0 / jnp.maximum(deg, 1.0)
  agg = (a0_ref[0].astype(jnp.float32) + a1_ref[0].astype(jnp.float32)) * inv
  h = jnp.maximum(s_ref[...] + _dot(agg, wn_ref[...]), 0.0)
  o_ref[...] = h
  ob_ref[...] = h.astype(jnp.bfloat16)
  inv_ref[...] = inv


def _layer2_body(s_ref, a0_ref, a1_ref, inv_ref, wn_ref, o_ref):
  agg = (a0_ref[0].astype(jnp.float32)
         + a1_ref[0].astype(jnp.float32)) * inv_ref[...]
  y = jnp.maximum(s_ref[...] + _dot(agg, wn_ref[...]), 0.0)
  nrm = jnp.sqrt(jnp.sum(y * y, axis=1, keepdims=True))
  o_ref[...] = y / jnp.maximum(nrm, 1e-12)


_BR = 1000
_row = pl.BlockSpec((_BR, D), lambda i: (i, 0))
_col1 = pl.BlockSpec((_BR, 1), lambda i: (i, 0))
_acc0 = pl.BlockSpec((1, _BR, D), lambda i: (0, i, 0))
_acc1 = pl.BlockSpec((1, _BR, D), lambda i: (1, i, 0))
_degs = pl.BlockSpec((2, _BR, 16), lambda i: (0, i, 0))
_wspec = pl.BlockSpec((D, D), lambda i: (0, 0))
_bspec = pl.BlockSpec((1, D), lambda i: (0, 0))

_tc_self = pl.pallas_call(
    _self_body,
    grid=(N // _BR,),
    in_specs=[_row, _wspec, _bspec],
    out_specs=_row,
    out_shape=jax.ShapeDtypeStruct((N, D), jnp.float32),
)

_tc_layer1 = pl.pallas_call(
    _layer1_body,
    grid=(N // _BR,),
    in_specs=[_row, _acc0, _acc1, _degs, _wspec],
    out_specs=[_row, _row, _col1],
    out_shape=[jax.ShapeDtypeStruct((N, D), jnp.float32),
               jax.ShapeDtypeStruct((N, D), jnp.bfloat16),
               jax.ShapeDtypeStruct((N, 1), jnp.float32)],
)

_tc_layer2 = pl.pallas_call(
    _layer2_body,
    grid=(N // _BR,),
    in_specs=[_row, _acc0, _acc1, _col1, _wspec],
    out_specs=_row,
    out_shape=jax.ShapeDtypeStruct((N, D), jnp.float32),
)


@jax.jit
def kernel(x, edge_index, W_self1, W_neigh1, b1, W_self2, W_neigh2, b2):
  src = edge_index[0].astype(jnp.int32)
  dst = edge_index[1].astype(jnp.int32)
  pidx = (src | (dst << 16)).reshape(NW, EPW // KB, KB)
  acc1, degp = _sc_agg_deg(x.astype(jnp.bfloat16), pidx)
  s1 = _tc_self(x, W_self1, b1.reshape(1, D))
  h1, h1b, inv = _tc_layer1(s1, acc1, acc1, degp, W_neigh1)
  (acc2,) = _sc_agg(h1b, pidx)
  s2 = _tc_self(h1, W_self2, b2.reshape(1, D))
  return _tc_layer2(s2, acc2, acc2, inv, W_neigh2)


# speedup vs baseline: 1.7345x; 1.0050x over previous
import functools

import jax
import jax.numpy as jnp
from jax import lax
from jax.experimental import pallas as pl
from jax.experimental.pallas import tpu as pltpu
from jax.experimental.pallas import tpu_sc as plsc

N = 10000
D = 128
E = 320000
NC = 2
NS = 16
NW = NC * NS
EPW = E // NW
K = 40
NB = EPW // K
NP = N
RPT = NP // NS
DZ = 25


def _sc_agg_build(with_deg, k, dtype):
  nb = EPW // k
  assert nb * k == EPW and k % 16 == 0 and k <= 128 and D % 32 == 0
  RB = 8 if dtype == jnp.bfloat16 else 2
  IS = 2 * RB
  lanes = 16 if dtype == jnp.float32 else 32
  mesh = plsc.VectorSubcoreMesh(core_axis_name="c", subcore_axis_name="s")
  out_type = [jax.ShapeDtypeStruct((NC, NP, D), dtype)]
  scratch = [
      pltpu.VMEM((nb, k), jnp.int32),
      pltpu.VMEM((IS, k), jnp.int32),
      pltpu.VMEM((IS, k), jnp.int32),
  ] + [pltpu.VMEM((k, D), dtype) for _ in range(RB)] + [
      pltpu.VMEM_SHARED((NP, D), dtype),
  ] + [pltpu.SemaphoreType.DMA for _ in range(RB)] + [
      pltpu.SemaphoreType.DMA,
  ]
  if with_deg:
    out_type.append(jax.ShapeDtypeStruct((NC, NP, 16), jnp.float32))
    scratch += [
        pltpu.VMEM((k, 16), jnp.float32),
        pltpu.VMEM_SHARED((NP, 16), jnp.float32),
        pltpu.VMEM((DZ, 16), jnp.float32),
    ]

  def body(p_hbm, pidx_hbm, *rest):
    rest = list(rest)
    acc_o = rest.pop(0)
    if with_deg:
      deg_o = rest.pop(0)
    pidx_v, sidx_v, didx_v = rest[0], rest[1], rest[2]
    rows = rest[3:3 + RB]
    acc_sh = rest[3 + RB]
    sems = rest[4 + RB:4 + 2 * RB]
    osem = rest[4 + 2 * RB]
    if with_deg:
      ones_v, deg_sh, dzb_v = rest[5 + 2 * RB:8 + 2 * RB]
    cid = lax.axis_index("c")
    sid = lax.axis_index("s")
    wid = cid * NS + sid

    rows_v = rows[0]
    def zfill(i, _):
      for c in range(D // lanes):
        rows_v[i, pl.ds(c * lanes, lanes)] = jnp.zeros((lanes,), dtype)
      return 0
    lax.fori_loop(0, k, zfill, 0)
    for j in range(RPT // k):
      pltpu.sync_copy(rows_v, acc_sh.at[pl.ds(sid * RPT + j * k, k)])
    rem = RPT % k
    if rem:
      pltpu.sync_copy(rows_v.at[pl.ds(0, rem)],
                      acc_sh.at[pl.ds(sid * RPT + (RPT // k) * k, rem)])
    if with_deg:
      def dzfill(i, _):
        dzb_v[i, :] = jnp.zeros((16,), jnp.float32)
        return 0
      lax.fori_loop(0, DZ, dzfill, 0)
      for j in range(RPT // DZ):
        pltpu.sync_copy(dzb_v, deg_sh.at[pl.ds(sid * RPT + j * DZ, DZ)])
      def ofill(i, _):
        ones_v[i, :] = jnp.ones((16,), jnp.float32)
        return 0
      lax.fori_loop(0, k, ofill, 0)
    plsc.subcore_barrier()

    pltpu.sync_copy(pidx_hbm.at[wid], pidx_v)


    def unpack(i, s):
      for c in range(k // 16):
        v = pidx_v[i, pl.ds(c * 16, 16)]
        sidx_v[s, pl.ds(c * 16, 16)] = v & 0xFFFF
        didx_v[s, pl.ds(c * 16, 16)] = lax.shift_right_logical(v, 16)

    def start(b, s):
      pltpu.async_copy(p_hbm.at[sidx_v.at[s]], rows[b], sems[b])

    def finish(b, s, nxt=None, nxt_s=None):
      pltpu.make_async_copy(p_hbm.at[sidx_v.at[s]], rows[b], sems[b]).wait()
      pltpu.sync_copy(rows[b], acc_sh.at[didx_v.at[s]], add=True)
      if with_deg:
        pltpu.async_copy(ones_v, deg_sh.at[didx_v.at[s]], osem, add=True)
      if nxt is not None:
        unpack(nxt, nxt_s)
        start(b, nxt_s)

    assert IS % RB == 0 and nb > IS
    G = (nb - RB) // IS
    assert nb - IS * G <= IS
    for i in range(RB):
      unpack(i, i)
      start(i, i)

    def step(g, _):
      i0 = IS * g
      for j in range(IS):
        finish(j % RB, j, i0 + j + RB, (j + RB) % IS)
      return 0
    lax.fori_loop(0, G, step, 0)
    for i in range(IS * G, nb):
      j = i - IS * G
      if i + RB < nb:
        finish(j % RB, j % IS, i + RB, (j + RB) % IS)
      else:
        finish(j % RB, j % IS)
    if with_deg:
      def odrain(i, _):
        pltpu.make_async_copy(ones_v, deg_sh.at[didx_v.at[0]], osem).wait()
        return 0
      lax.fori_loop(0, nb, odrain, 0)
    plsc.subcore_barrier()

    r0 = sid * RPT
    pltpu.sync_copy(acc_sh.at[pl.ds(r0, RPT)],
                    acc_o.at[cid, pl.ds(r0, RPT)])
    if with_deg:
      pltpu.sync_copy(deg_sh.at[pl.ds(r0, RPT)],
                      deg_o.at[cid, pl.ds(r0, RPT)])

  return pl.kernel(
      body, out_type=out_type, mesh=mesh, scratch_types=scratch,
      compiler_params=pltpu.CompilerParams(use_tc_tiling_on_sc=False))


KB = 80
_sc_agg_deg = _sc_agg_build(True, KB, jnp.bfloat16)
_sc_agg = _sc_agg_build(False, KB, jnp.bfloat16)


def _dot(a, b):
  return jnp.dot(a, b, preferred_element_type=jnp.float32,
                 precision=lax.Precision.HIGHEST)


def _self_body(x_ref, w_ref, b_ref, o_ref):
  o_ref[...] = _dot(x_ref[...], w_ref[...]) + b_ref[...]


def _layer1_body(s_ref, a0_ref, a1_ref, deg_ref, wn_ref,
                 o_ref, ob_ref, inv_ref):
  deg = jnp.sum(deg_ref[0] + deg_ref[1], axis=1, keepdims=True) * (1.0 / 16.0)
  inv = 1.0 / jnp.maximum(deg, 1.0)
  agg = (a0_ref[0].astype(jnp.float32) + a1_ref[0].astype(jnp.float32)) * inv
  h = jnp.maximum(s_ref[...] + _dot(agg, wn_ref[...]), 0.0)
  o_ref[...] = h
  ob_ref[...] = h.astype(jnp.bfloat16)
  inv_ref[...] = inv


def _layer2_body(s_ref, a0_ref, a1_ref, inv_ref, wn_ref, o_ref):
  agg = (a0_ref[0].astype(jnp.float32)
         + a1_ref[0].astype(jnp.float32)) * inv_ref[...]
  y = jnp.maximum(s_ref[...] + _dot(agg, wn_ref[...]), 0.0)
  nrm = jnp.sqrt(jnp.sum(y * y, axis=1, keepdims=True))
  o_ref[...] = y / jnp.maximum(nrm, 1e-12)


_BR = 1000
_row = pl.BlockSpec((_BR, D), lambda i: (i, 0))
_col1 = pl.BlockSpec((_BR, 1), lambda i: (i, 0))
_acc0 = pl.BlockSpec((1, _BR, D), lambda i: (0, i, 0))
_acc1 = pl.BlockSpec((1, _BR, D), lambda i: (1, i, 0))
_degs = pl.BlockSpec((2, _BR, 16), lambda i: (0, i, 0))
_wspec = pl.BlockSpec((D, D), lambda i: (0, 0))
_bspec = pl.BlockSpec((1, D), lambda i: (0, 0))

_tc_self = pl.pallas_call(
    _self_body,
    grid=(N // _BR,),
    in_specs=[_row, _wspec, _bspec],
    out_specs=_row,
    out_shape=jax.ShapeDtypeStruct((N, D), jnp.float32),
)

_tc_layer1 = pl.pallas_call(
    _layer1_body,
    grid=(N // _BR,),
    in_specs=[_row, _acc0, _acc1, _degs, _wspec],
    out_specs=[_row, _row, _col1],
    out_shape=[jax.ShapeDtypeStruct((N, D), jnp.float32),
               jax.ShapeDtypeStruct((N, D), jnp.bfloat16),
               jax.ShapeDtypeStruct((N, 1), jnp.float32)],
)

_tc_layer2 = pl.pallas_call(
    _layer2_body,
    grid=(N // _BR,),
    in_specs=[_row, _acc0, _acc1, _col1, _wspec],
    out_specs=_row,
    out_shape=jax.ShapeDtypeStruct((N, D), jnp.float32),
)


@jax.jit
def kernel(x, edge_index, W_self1, W_neigh1, b1, W_self2, W_neigh2, b2):
  src = edge_index[0].astype(jnp.int32)
  dst = edge_index[1].astype(jnp.int32)
  pidx = (src | (dst << 16)).reshape(NW, EPW // KB, KB)
  acc1, degp = _sc_agg_deg(x.astype(jnp.bfloat16), pidx)
  s1 = _tc_self(x, W_self1, b1.reshape(1, D))
  h1, h1b, inv = _tc_layer1(s1, acc1, acc1, degp, W_neigh1)
  (acc2,) = _sc_agg(h1b, pidx)
  s2 = _tc_self(h1, W_self2, b2.reshape(1, D))
  return _tc_layer2(s2, acc2, acc2, inv, W_neigh2)


# speedup vs baseline: 1.7368x; 1.0013x over previous
import jax
import jax.numpy as jnp
from jax import lax
from jax.experimental import pallas as pl
from jax.experimental.pallas import tpu as pltpu
from jax.experimental.pallas import tpu_sc as plsc

N = 10000
D = 128
E = 320000
NC = 2
NS = 16
NW = NC * NS
EPW = E // NW
NP = N
RPT = NP // NS
DZ = 25


def _sc_agg_build(with_deg, k, dtype):
  nb = EPW // k
  assert nb * k == EPW and k % 16 == 0 and k <= 128 and D % 32 == 0
  RB = 8 if dtype == jnp.bfloat16 else 2
  IS = 2 * RB
  lanes = 16 if dtype == jnp.float32 else 32
  mesh = plsc.VectorSubcoreMesh(core_axis_name="c", subcore_axis_name="s")
  out_type = [jax.ShapeDtypeStruct((NC, NP, D), dtype)]
  scratch = [
      pltpu.VMEM((nb, k), jnp.int32),
      pltpu.VMEM((IS, k), jnp.int32),
      pltpu.VMEM((IS, k), jnp.int32),
  ] + [pltpu.VMEM((k, D), dtype) for _ in range(RB)] + [
      pltpu.VMEM_SHARED((NP, D), dtype),
  ] + [pltpu.SemaphoreType.DMA for _ in range(RB)] + [
      pltpu.SemaphoreType.DMA,
  ]
  if with_deg:
    out_type.append(jax.ShapeDtypeStruct((NC, NP, 16), jnp.float32))
    scratch += [
        pltpu.VMEM((k, 16), jnp.float32),
        pltpu.VMEM_SHARED((NP, 16), jnp.float32),
        pltpu.VMEM((DZ, 16), jnp.float32),
    ]

  def body(p_hbm, pidx_hbm, *rest):
    rest = list(rest)
    acc_o = rest.pop(0)
    if with_deg:
      deg_o = rest.pop(0)
    pidx_v, sidx_v, didx_v = rest[0], rest[1], rest[2]
    rows = rest[3:3 + RB]
    acc_sh = rest[3 + RB]
    sems = rest[4 + RB:4 + 2 * RB]
    osem = rest[4 + 2 * RB]
    if with_deg:
      ones_v, deg_sh, dzb_v = rest[5 + 2 * RB:8 + 2 * RB]
    cid = lax.axis_index("c")
    sid = lax.axis_index("s")
    wid = cid * NS + sid

    rows_v = rows[0]
    def zfill(i, _):
      for c in range(D // lanes):
        rows_v[i, pl.ds(c * lanes, lanes)] = jnp.zeros((lanes,), dtype)
      return 0
    lax.fori_loop(0, k, zfill, 0)
    for j in range(RPT // k):
      pltpu.sync_copy(rows_v, acc_sh.at[pl.ds(sid * RPT + j * k, k)])
    rem = RPT % k
    if rem:
      pltpu.sync_copy(rows_v.at[pl.ds(0, rem)],
                      acc_sh.at[pl.ds(sid * RPT + (RPT // k) * k, rem)])
    if with_deg:
      def dzfill(i, _):
        dzb_v[i, :] = jnp.zeros((16,), jnp.float32)
        return 0
      lax.fori_loop(0, DZ, dzfill, 0)
      for j in range(RPT // DZ):
        pltpu.sync_copy(dzb_v, deg_sh.at[pl.ds(sid * RPT + j * DZ, DZ)])
      def ofill(i, _):
        ones_v[i, :] = jnp.ones((16,), jnp.float32)
        return 0
      lax.fori_loop(0, k, ofill, 0)
    plsc.subcore_barrier()

    pltpu.sync_copy(pidx_hbm.at[wid], pidx_v)


    def unpack(i, s):
      for c in range(k // 16):
        v = pidx_v[i, pl.ds(c * 16, 16)]
        sidx_v[s, pl.ds(c * 16, 16)] = v & 0xFFFF
        didx_v[s, pl.ds(c * 16, 16)] = lax.shift_right_logical(v, 16)

    def start(b, s):
      pltpu.async_copy(p_hbm.at[sidx_v.at[s]], rows[b], sems[b])

    def finish(b, s, nxt=None, nxt_s=None):
      pltpu.make_async_copy(p_hbm.at[sidx_v.at[s]], rows[b], sems[b]).wait()
      pltpu.sync_copy(rows[b], acc_sh.at[didx_v.at[s]], add=True)
      if with_deg:
        pltpu.async_copy(ones_v, deg_sh.at[didx_v.at[s]], osem, add=True)
      if nxt is not None:
        unpack(nxt, nxt_s)
        start(b, nxt_s)

    assert IS % RB == 0 and nb > IS
    G = (nb - RB) // IS
    assert nb - IS * G <= IS
    for i in range(RB):
      unpack(i, i)
      start(i, i)

    def step(g, _):
      i0 = IS * g
      for j in range(IS):
        finish(j % RB, j, i0 + j + RB, (j + RB) % IS)
      return 0
    lax.fori_loop(0, G, step, 0)
    for i in range(IS * G, nb):
      j = i - IS * G
      if i + RB < nb:
        finish(j % RB, j % IS, i + RB, (j + RB) % IS)
      else:
        finish(j % RB, j % IS)
    if with_deg:
      def odrain(i, _):
        pltpu.make_async_copy(ones_v, deg_sh.at[didx_v.at[0]], osem).wait()
        return 0
      lax.fori_loop(0, nb, odrain, 0)
    plsc.subcore_barrier()

    r0 = sid * RPT
    pltpu.sync_copy(acc_sh.at[pl.ds(r0, RPT)],
                    acc_o.at[cid, pl.ds(r0, RPT)])
    if with_deg:
      pltpu.sync_copy(deg_sh.at[pl.ds(r0, RPT)],
                      deg_o.at[cid, pl.ds(r0, RPT)])

  return pl.kernel(
      body, out_type=out_type, mesh=mesh, scratch_types=scratch,
      compiler_params=pltpu.CompilerParams(use_tc_tiling_on_sc=False))


KB = 80
_sc_agg_deg = _sc_agg_build(True, KB, jnp.bfloat16)
_sc_agg = _sc_agg_build(False, KB, jnp.bfloat16)


def _dot(a, b):
  return jnp.dot(a, b, preferred_element_type=jnp.float32,
                 precision=lax.Precision.HIGHEST)


def _self_body(x_ref, w_ref, b_ref, o_ref):
  o_ref[...] = _dot(x_ref[...], w_ref[...]) + b_ref[...]


def _layer1_body(s_ref, a0_ref, a1_ref, deg_ref, wn_ref,
                 o_ref, ob_ref, inv_ref):
  deg = jnp.sum(deg_ref[0] + deg_ref[1], axis=1, keepdims=True) * (1.0 / 16.0)
  inv = 1.0 / jnp.maximum(deg, 1.0)
  agg = (a0_ref[0].astype(jnp.float32) + a1_ref[0].astype(jnp.float32)) * inv
  h = jnp.maximum(s_ref[...] + _dot(agg, wn_ref[...]), 0.0)
  o_ref[...] = h
  ob_ref[...] = h.astype(jnp.bfloat16)
  inv_ref[...] = inv


def _layer2_body(s_ref, a0_ref, a1_ref, inv_ref, wn_ref, o_ref):
  agg = (a0_ref[0].astype(jnp.float32)
         + a1_ref[0].astype(jnp.float32)) * inv_ref[...]
  y = jnp.maximum(s_ref[...] + _dot(agg, wn_ref[...]), 0.0)
  nrm = jnp.sqrt(jnp.sum(y * y, axis=1, keepdims=True))
  o_ref[...] = y / jnp.maximum(nrm, 1e-12)


_BR = 1000
_row = pl.BlockSpec((_BR, D), lambda i: (i, 0))
_col1 = pl.BlockSpec((_BR, 1), lambda i: (i, 0))
_acc0 = pl.BlockSpec((1, _BR, D), lambda i: (0, i, 0))
_acc1 = pl.BlockSpec((1, _BR, D), lambda i: (1, i, 0))
_degs = pl.BlockSpec((2, _BR, 16), lambda i: (0, i, 0))
_wspec = pl.BlockSpec((D, D), lambda i: (0, 0))
_bspec = pl.BlockSpec((1, D), lambda i: (0, 0))

_tc_self = pl.pallas_call(
    _self_body,
    grid=(N // _BR,),
    in_specs=[_row, _wspec, _bspec],
    out_specs=_row,
    out_shape=jax.ShapeDtypeStruct((N, D), jnp.float32),
)

_tc_layer1 = pl.pallas_call(
    _layer1_body,
    grid=(N // _BR,),
    in_specs=[_row, _acc0, _acc1, _degs, _wspec],
    out_specs=[_row, _row, _col1],
    out_shape=[jax.ShapeDtypeStruct((N, D), jnp.float32),
               jax.ShapeDtypeStruct((N, D), jnp.bfloat16),
               jax.ShapeDtypeStruct((N, 1), jnp.float32)],
)

_tc_layer2 = pl.pallas_call(
    _layer2_body,
    grid=(N // _BR,),
    in_specs=[_row, _acc0, _acc1, _col1, _wspec],
    out_specs=_row,
    out_shape=jax.ShapeDtypeStruct((N, D), jnp.float32),
)


@jax.jit
def kernel(x, edge_index, W_self1, W_neigh1, b1, W_self2, W_neigh2, b2):
  src = edge_index[0].astype(jnp.int32)
  dst = edge_index[1].astype(jnp.int32)
  pidx = (src | (dst << 16)).reshape(NW, EPW // KB, KB)
  acc1, degp = _sc_agg_deg(x.astype(jnp.bfloat16), pidx)
  s1 = _tc_self(x, W_self1, b1.reshape(1, D))
  h1, h1b, inv = _tc_layer1(s1, acc1, acc1, degp, W_neigh1)
  (acc2,) = _sc_agg(h1b, pidx)
  s2 = _tc_self(h1, W_self2, b2.reshape(1, D))
  return _tc_layer2(s2, acc2, acc2, inv, W_neigh2)
